# baseline (speedup 1.0000x reference)
import functools

import jax
import jax.numpy as jnp
from jax import lax
from jax.experimental import pallas as pl
from jax.experimental.pallas import tpu as pltpu
from jax.experimental.pallas import tpu_sc as plsc

_DEBUG_JNP_PASS2 = False
_DEBUG_JNP_PASS1 = False

NP = 10240
BN = 512
BE_TC = 640
BE = 64
NC, NS, LANES = 2, 16, 16
NW = NC * NS


def _f32(*shape):
    return jax.ShapeDtypeStruct(shape, jnp.float32)


def _proj(x, wl, wr, c_chunks):
    n, k = x.shape
    d = wl.shape[1]

    def body(x_ref, wl_ref, wr_ref, xlc_ref, xr_ref, xlcb_ref, xrb_ref):
        xb = x_ref[...]
        xl = jnp.dot(xb, wl_ref[...], preferred_element_type=jnp.float32)
        xr = jnp.dot(xb, wr_ref[...], preferred_element_type=jnp.float32)
        xr_ref[...] = xr
        xrb_ref[...] = xr.astype(jnp.bfloat16)
        for c in range(c_chunks):
            xlc_ref[c] = xl[:, c * 128:(c + 1) * 128]
        for c2 in range(c_chunks // 2):
            xlcb_ref[c2] = xl[:, c2 * 256:(c2 + 1) * 256].astype(jnp.bfloat16)

    return pl.pallas_call(
        body,
        grid=(n // BN,),
        in_specs=[
            pl.BlockSpec((BN, k), lambda i: (i, 0)),
            pl.BlockSpec((k, d), lambda i: (0, 0)),
            pl.BlockSpec((k, d), lambda i: (0, 0)),
        ],
        out_specs=[
            pl.BlockSpec((c_chunks, BN, 128), lambda i: (0, i, 0)),
            pl.BlockSpec((BN, d), lambda i: (i, 0)),
            pl.BlockSpec((c_chunks // 2, BN, 256), lambda i: (0, i, 0)),
            pl.BlockSpec((BN, d), lambda i: (i, 0)),
        ],
        out_shape=[_f32(c_chunks, n, 128), _f32(n, d),
                   jax.ShapeDtypeStruct((c_chunks // 2, n, 256), jnp.bfloat16),
                   jax.ShapeDtypeStruct((n, d), jnp.bfloat16)],
    )(x, wl, wr)


def _edge_transform(ea, we):
    e, de = ea.shape
    d = we.shape[1]

    def body(ea_ref, we_ref, ew_ref):
        ew_ref[...] = jnp.dot(ea_ref[...], we_ref[...],
                              preferred_element_type=jnp.float32
                              ).astype(jnp.bfloat16)

    return pl.pallas_call(
        body,
        grid=(e // BE_TC,),
        in_specs=[
            pl.BlockSpec((BE_TC, de), lambda i: (i, 0)),
            pl.BlockSpec((de, d), lambda i: (0, 0)),
        ],
        out_specs=pl.BlockSpec((BE_TC, d), lambda i: (i, 0)),
        out_shape=jax.ShapeDtypeStruct((e, d), jnp.bfloat16),
    )(ea, we)


def _loop_attr(attr_p, deg_p, de):
    def body(a_ref, d_ref, o_ref):
        deg = jnp.sum(d_ref[...], axis=0)
        asum = jnp.sum(a_ref[...], axis=0)[:, :de]
        o_ref[...] = asum / jnp.maximum(deg, 1.0)[:, None]

    return pl.pallas_call(
        body,
        grid=(NP // BN,),
        in_specs=[
            pl.BlockSpec((2, BN, 128), lambda i: (0, i, 0)),
            pl.BlockSpec((NW, BN), lambda i: (0, i)),
        ],
        out_specs=pl.BlockSpec((BN, de), lambda i: (i, 0)),
        out_shape=_f32(NP, de),
    )(attr_p, deg_p)


def _p_self(xlc, xr, la, we, att2d, c_chunks):
    d = xr.shape[1]
    de = la.shape[1]

    def body(xlc_ref, xr_ref, la_ref, we_ref, att_ref, o_ref):
        lw = jnp.dot(la_ref[...], we_ref[...],
                     preferred_element_type=jnp.float32)
        acc = jnp.zeros((BN,), jnp.float32)
        for c in range(c_chunks):
            sl = slice(c * 128, (c + 1) * 128)
            v = xlc_ref[c] + xr_ref[:, sl] + lw[:, sl]
            v = jnp.maximum(v, 0.2 * v)
            acc = acc + jnp.sum(v * att_ref[0, sl][None, :], axis=1)
        o_ref[...] = jnp.exp(acc)[:, None]

    return pl.pallas_call(
        body,
        grid=(NP // BN,),
        in_specs=[
            pl.BlockSpec((c_chunks, BN, 128), lambda i: (0, i, 0)),
            pl.BlockSpec((BN, d), lambda i: (i, 0)),
            pl.BlockSpec((BN, de), lambda i: (i, 0)),
            pl.BlockSpec((de, d), lambda i: (0, 0)),
            pl.BlockSpec((1, d), lambda i: (0, 0)),
        ],
        out_specs=pl.BlockSpec((BN, 1), lambda i: (i, 0)),
        out_shape=_f32(NP, 1),
    )(xlc, xr, la, we, att2d)


def _combine(ou_list, xlc, s_p, p_self, b2d, c_chunks, relu):
    d = c_chunks * 128

    def body(*refs):
        ou_refs = refs[:c_chunks]
        xlc_ref, s_ref, ps_ref, b_ref, o_ref = refs[c_chunks:]
        den = jnp.sum(s_ref[...], axis=0)[:, None] + ps_ref[...] + 1e-16
        for c in range(c_chunks):
            num = ou_refs[c][0] + ou_refs[c][1] + ps_ref[...] * xlc_ref[c]
            val = num / den + b_ref[0, c * 128:(c + 1) * 128][None, :]
            if relu:
                val = jnp.maximum(val, 0.0)
            o_ref[:, c * 128:(c + 1) * 128] = val

    return pl.pallas_call(
        body,
        grid=(NP // BN,),
        in_specs=[pl.BlockSpec((2, BN, 128), lambda i: (0, i, 0))
                  for _ in range(c_chunks)] + [
            pl.BlockSpec((c_chunks, BN, 128), lambda i: (0, i, 0)),
            pl.BlockSpec((NW, BN), lambda i: (0, i)),
            pl.BlockSpec((BN, 1), lambda i: (i, 0)),
            pl.BlockSpec((1, d), lambda i: (0, 0)),
        ],
        out_specs=pl.BlockSpec((BN, d), lambda i: (i, 0)),
        out_shape=_f32(NP, d),
    )(*ou_list, xlc, s_p, p_self, b2d)


def _zero_1d(ref, n):
    def zb(i, _):
        ref[pl.ds(i * LANES, LANES)] = jnp.zeros((LANES,), jnp.float32)
        return 0
    lax.fori_loop(0, n // LANES, zb, 0)


def _edge_blocks(nblocks, w, body_fn):
    kmax = -(-nblocks // NW)

    def blk(kk, _):
        b = kk * NW + w

        @pl.when(b < nblocks)
        def _():
            body_fn(b * BE)
        return 0

    lax.fori_loop(0, kmax, blk, 0)


def _make_pass0(e, de):
    mesh = plsc.VectorSubcoreMesh(core_axis_name="c", subcore_axis_name="s", num_cores=NC, num_subcores=NS)
    rows_per_tile = NP // NS
    zrows = 32

    @functools.partial(
        pl.kernel,
        out_type=(_f32(2, NP, 128), _f32(NW, NP)),
        mesh=mesh,
        compiler_params=pltpu.CompilerParams(needs_layout_passes=False),
        scratch_types=(
            pltpu.VMEM((BE,), jnp.int32),
            pltpu.VMEM((BE, de), jnp.float32),
            pltpu.VMEM((BE, 128), jnp.float32),
            pltpu.VMEM((NP,), jnp.float32),
            pltpu.VMEM((zrows, 128), jnp.float32),
            pltpu.VMEM_SHARED((NP, 128), jnp.float32),
            pltpu.SemaphoreType.DMA,
        ),
    )
    def k(dst_h, ea_h, attr_h, deg_h, dstb, eab, rows, dega, ztile, acc, sem):
        c_ax = lax.axis_index("c")
        s_ax = lax.axis_index("s")
        w = s_ax * NC + c_ax

        _zero_1d(dega, NP)

        def zrow(i, _):
            for j in range(128 // LANES):
                ztile[i, pl.ds(j * LANES, LANES)] = jnp.zeros(
                    (LANES,), jnp.float32)
            return 0
        lax.fori_loop(0, zrows, zrow, 0)

        def zrow2(i, _):
            for j in range(128 // LANES):
                rows[i, pl.ds(j * LANES, LANES)] = jnp.zeros(
                    (LANES,), jnp.float32)
            return 0
        lax.fori_loop(0, BE, zrow2, 0)

        def zcp(q, _):
            pltpu.sync_copy(
                ztile,
                acc.at[pl.ds(s_ax * rows_per_tile + q * zrows, zrows)])
            return 0
        lax.fori_loop(0, rows_per_tile // zrows, zcp, 0)
        plsc.subcore_barrier()

        def do_block(base):
            pltpu.sync_copy(dst_h.at[pl.ds(base, BE)], dstb)
            pltpu.sync_copy(ea_h.at[pl.ds(base, BE)], eab)

            def crow(i, _):
                rows[i, pl.ds(0, LANES)] = eab[i, pl.ds(0, LANES)]
                return 0
            lax.fori_loop(0, BE, crow, 0)
            pltpu.sync_copy(rows, acc.at[dstb], add=True)
            ones = jnp.ones((LANES,), jnp.float32)
            for jj in range(BE // LANES):
                idx = dstb[pl.ds(jj * LANES, LANES)]
                plsc.addupdate_scatter(dega, [idx], ones)

        _edge_blocks(e // BE, w, do_block)

        pltpu.sync_copy(dega, deg_h.at[w])
        plsc.subcore_barrier()
        sl = pl.ds(s_ax * rows_per_tile, rows_per_tile)
        pltpu.sync_copy(acc.at[sl], attr_h.at[c_ax, sl])

    return k


def _make_pass1(e, d, c_chunks):
    mesh = plsc.VectorSubcoreMesh(core_axis_name="c", subcore_axis_name="s", num_cores=NC, num_subcores=NS)
    be = 64
    nb = e // be
    kmax = -(-nb // NW)
    c2n = c_chunks // 2

    buf_set = tuple(
        (pltpu.VMEM((be,), jnp.int32),
         pltpu.VMEM((be,), jnp.int32),
         pltpu.VMEM((be, d // 2), jnp.int32),
         pltpu.VMEM((be, d // 2), jnp.int32),
         tuple(pltpu.VMEM((be, 128), jnp.int32)
               for _ in range(c2n)),
         pltpu.SemaphoreType.DMA)
        for _ in range(2))

    @functools.partial(
        pl.kernel,
        out_type=(_f32(e), _f32(NW, NP)),
        mesh=mesh,
        compiler_params=pltpu.CompilerParams(needs_layout_passes=False),
        scratch_types=(
            buf_set,
            pltpu.VMEM((d,), jnp.float32),
            pltpu.VMEM((be,), jnp.float32),
            pltpu.VMEM((be,), jnp.float32),
            pltpu.VMEM((NP,), jnp.float32),
        ),
    )
    def k(src_h, dst_h, xr_h, ew_h, att_h, *rest):
        xl_hs = rest[:c2n]
        p_h, s_h = rest[c2n:c2n + 2]
        bufs, attv, lblock, pblock, sacc = rest[c2n + 2:]

        c_ax = lax.axis_index("c")
        s_ax = lax.axis_index("s")
        w = s_ax * NC + c_ax

        pltpu.sync_copy(att_h, attv)
        _zero_1d(sacc, NP)

        lane_iota = lax.iota(jnp.int32, LANES)

        def issue(t, b):
            srcb, dstb, ewb, xrb, xlbs, sem = bufs[t]
            base = b * be
            pltpu.sync_copy(src_h.at[pl.ds(base, be)], srcb)
            pltpu.sync_copy(dst_h.at[pl.ds(base, be)], dstb)
            pltpu.async_copy(ew_h.at[pl.ds(base, be)], ewb, sem)
            pltpu.async_copy(xr_h.at[dstb], xrb, sem)
            for c2 in range(c2n):
                pltpu.async_copy(xl_hs[c2].at[srcb], xlbs[c2], sem)

        def compute(t, b):
            srcb, dstb, ewb, xrb, xlbs, sem = bufs[t]
            base = b * be
            pltpu.make_async_copy(ew_h.at[pl.ds(0, be)], ewb, sem).wait()
            pltpu.make_async_copy(xr_h.at[dstb], xrb, sem).wait()
            for c2 in range(c2n):
                pltpu.make_async_copy(xl_hs[c2].at[srcb], xlbs[c2],
                                      sem).wait()

            himask = jnp.full((LANES,), -65536, jnp.int32)

            def lo16(x):
                return plsc.bitcast(lax.shift_left(x, 16), jnp.float32)

            def hi16(x):
                return plsc.bitcast(jnp.bitwise_and(x, himask), jnp.float32)

            def edge(i, lvec):
                acc = jnp.zeros((LANES,), jnp.float32)
                for c2 in range(c2n):
                    for j2 in range(256 // 32):
                        f = c2 * 256 + j2 * 32
                        xl32 = xlbs[c2][i, pl.ds(j2 * LANES, LANES)]
                        xr32 = xrb[i, pl.ds(f // 2, LANES)]
                        ew32 = ewb[i, pl.ds(f // 2, LANES)]
                        va = lo16(xl32) + lo16(xr32) + lo16(ew32)
                        va = jnp.maximum(va, 0.2 * va)
                        acc = acc + va * attv[pl.ds(f, LANES)]
                        vb = hi16(xl32) + hi16(xr32) + hi16(ew32)
                        vb = jnp.maximum(vb, 0.2 * vb)
                        acc = acc + vb * attv[pl.ds(f + LANES, LANES)]
                lsum = jnp.sum(acc)
                lvec = jnp.where(lane_iota == i % LANES, lsum, lvec)

                @pl.when(i % LANES == LANES - 1)
                def _():
                    lblock[pl.ds(i - (LANES - 1), LANES)] = lvec
                return lvec

            lax.fori_loop(0, be, edge, jnp.zeros((LANES,), jnp.float32))

            for jj in range(be // LANES):
                pv = jnp.exp(lblock[pl.ds(jj * LANES, LANES)])
                pblock[pl.ds(jj * LANES, LANES)] = pv
                idx = dstb[pl.ds(jj * LANES, LANES)]
                plsc.addupdate_scatter(sacc, [idx], pv)
            pltpu.sync_copy(pblock, p_h.at[pl.ds(base, be)])

        def blk_of(j):
            return j * NW + w

        @pl.when(blk_of(0) < nb)
        def _():
            issue(0, blk_of(0))

        def pair(kk, _):
            j0 = 2 * kk

            @pl.when(blk_of(j0 + 1) < nb)
            def _():
                issue(1, blk_of(j0 + 1))

            @pl.when(blk_of(j0) < nb)
            def _():
                compute(0, blk_of(j0))

            @pl.when(blk_of(j0 + 2) < nb)
            def _():
                issue(0, blk_of(j0 + 2))

            @pl.when(blk_of(j0 + 1) < nb)
            def _():
                compute(1, blk_of(j0 + 1))
            return 0

        lax.fori_loop(0, (kmax + 1) // 2, pair, 0)
        pltpu.sync_copy(sacc, s_h.at[w])

    return k


def _make_pass2(e, c_chunks):
    mesh = plsc.VectorSubcoreMesh(core_axis_name="c", subcore_axis_name="s", num_cores=NC, num_subcores=NS)
    rows_per_tile = NP // NS
    zrows = 32

    @functools.partial(
        pl.kernel,
        out_type=tuple(_f32(2, NP, 128) for _ in range(c_chunks)),
        mesh=mesh,
        compiler_params=pltpu.CompilerParams(needs_layout_passes=False),
        scratch_types=(
            tuple((pltpu.VMEM((BE,), jnp.int32),
                   pltpu.VMEM((BE,), jnp.int32),
                   pltpu.VMEM((BE,), jnp.float32),
                   pltpu.VMEM((BE, 128), jnp.float32),
                   pltpu.SemaphoreType.DMA)
                  for _ in range(2)),
            pltpu.VMEM((zrows, 128), jnp.float32),
            pltpu.VMEM_SHARED((NP, 128), jnp.float32),
        ),
    )
    def k(src_h, dst_h, p_h, *rest):
        xl_hs = rest[:c_chunks]
        ou_hs = rest[c_chunks:2 * c_chunks]
        bufs, ztile, acc = rest[2 * c_chunks:]

        c_ax = lax.axis_index("c")
        s_ax = lax.axis_index("s")
        w = s_ax * NC + c_ax

        def zrow(i, _):
            for j in range(128 // LANES):
                ztile[i, pl.ds(j * LANES, LANES)] = jnp.zeros(
                    (LANES,), jnp.float32)
            return 0
        lax.fori_loop(0, zrows, zrow, 0)

        nb = e // BE
        kmax = -(-nb // NW)
        lane_iota = lax.iota(jnp.int32, LANES)

        def blk_of(j):
            return j * NW + w

        for c in range(c_chunks):
            def zcp(q, _):
                pltpu.sync_copy(
                    ztile,
                    acc.at[pl.ds(s_ax * rows_per_tile + q * zrows, zrows)])
                return 0
            lax.fori_loop(0, rows_per_tile // zrows, zcp, 0)
            plsc.subcore_barrier()

            def issue(t, b):
                srcb, dstb, pb, rows, sem = bufs[t]
                base = b * BE
                pltpu.sync_copy(src_h.at[pl.ds(base, BE)], srcb)
                pltpu.sync_copy(dst_h.at[pl.ds(base, BE)], dstb)
                pltpu.sync_copy(p_h.at[pl.ds(base, BE)], pb)
                pltpu.async_copy(xl_hs[c].at[srcb], rows, sem)

            def proc(t, b):
                srcb, dstb, pb, rows, sem = bufs[t]
                pltpu.make_async_copy(xl_hs[c].at[srcb], rows, sem).wait()

                def edge(i, _):
                    pv = plsc.load_gather(pb, [lane_iota * 0 + i])
                    for j in range(128 // LANES):
                        sl = pl.ds(j * LANES, LANES)
                        rows[i, sl] = rows[i, sl] * pv
                    return 0

                lax.fori_loop(0, BE, edge, 0)
                pltpu.sync_copy(rows, acc.at[dstb], add=True)

            @pl.when(blk_of(0) < nb)
            def _():
                issue(0, blk_of(0))

            def pair(kk, _):
                j0 = 2 * kk

                @pl.when(blk_of(j0 + 1) < nb)
                def _():
                    issue(1, blk_of(j0 + 1))

                @pl.when(blk_of(j0) < nb)
                def _():
                    proc(0, blk_of(j0))

                @pl.when(blk_of(j0 + 2) < nb)
                def _():
                    issue(0, blk_of(j0 + 2))

                @pl.when(blk_of(j0 + 1) < nb)
                def _():
                    proc(1, blk_of(j0 + 1))
                return 0

            lax.fori_loop(0, (kmax + 1) // 2, pair, 0)
            plsc.subcore_barrier()
            sl = pl.ds(s_ax * rows_per_tile, rows_per_tile)
            pltpu.sync_copy(acc.at[sl], ou_hs[c].at[c_ax, sl])
            plsc.subcore_barrier()

    return k


def _gat_layer(xin, src, dst, ea, la, wl, wr, we, att, b, relu):
    e = src.shape[0]
    d = wl.shape[1]
    c_chunks = d // 128

    xlc, xr, xlcb, xrbf = _proj(xin, wl, wr, c_chunks)
    ew = _edge_transform(ea, we)
    att2d = att.reshape(1, d)
    ps = _p_self(xlc, xr, la, we, att2d, c_chunks)
    att_perm = att.reshape(d // 32, 16, 2).transpose(0, 2, 1).reshape(-1)

    def as_i32(a):
        return lax.bitcast_convert_type(
            a.reshape(a.shape[0], -1, 2), jnp.int32)

    xl_list = [xlc[c] for c in range(c_chunks)]
    xlb_list = [as_i32(xlcb[c2]) for c2 in range(c_chunks // 2)]
    if _DEBUG_JNP_PASS1:
        xl_full = jnp.concatenate(xl_list, axis=1)
        v = xl_full[src] + xr[dst] + ew
        logit = jnp.sum(jnp.maximum(v, 0.2 * v) * att[None, :], axis=1)
        p = jnp.exp(logit)
        s_p = jnp.zeros((NW, NP), jnp.float32)
        s_p = s_p.at[0].set(jax.ops.segment_sum(p, dst, num_segments=NP))
    else:
        p, s_p = _make_pass1(e, d, c_chunks)(src, dst, as_i32(xrbf),
                                             as_i32(ew), att_perm, *xlb_list)
    if _DEBUG_JNP_PASS2:
        ou_list = []
        for c in range(c_chunks):
            ou = jax.ops.segment_sum(p[:, None] * xl_list[c][src], dst,
                                     num_segments=NP)
            ou_list.append(jnp.stack([ou, jnp.zeros_like(ou)]))
    else:
        ou_list = _make_pass2(e, c_chunks)(src, dst, p, *xl_list)
    if not isinstance(ou_list, (list, tuple)):
        ou_list = [ou_list]
    return _combine(list(ou_list), xlc, s_p, ps, b.reshape(1, d),
                    c_chunks, relu)


def kernel(x, edge_index, edge_attr, Wl1, Wr1, We1, att1, b1,
           Wl2, Wr2, We2, att2, b2):
    n = x.shape[0]
    e = edge_index.shape[1]
    src = edge_index[0].astype(jnp.int32)
    dst = edge_index[1].astype(jnp.int32)
    ea = edge_attr.astype(jnp.float32)
    xp = jnp.pad(x.astype(jnp.float32), ((0, NP - n), (0, 0)))

    attr_p, deg_p = _make_pass0(e, ea.shape[1])(dst, ea)
    la = _loop_attr(attr_p, deg_p, ea.shape[1])

    h = _gat_layer(xp, src, dst, ea, la, Wl1, Wr1, We1, att1, b1, relu=True)
    out = _gat_layer(h, src, dst, ea, la, Wl2, Wr2, We2, att2, b2, relu=False)
    return out[:n]


# speedup vs baseline: 1.8317x; 1.8317x over previous
import functools

import jax
import jax.numpy as jnp
from jax import lax
from jax.experimental import pallas as pl
from jax.experimental.pallas import tpu as pltpu
from jax.experimental.pallas import tpu_sc as plsc

_DEBUG_JNP_PASS2 = False
_DEBUG_JNP_PASS1 = False

NP = 10240
BN = 512
BE_TC = 640
BE = 64
NC, NS, LANES = 2, 16, 16
NW = NC * NS


def _f32(*shape):
    return jax.ShapeDtypeStruct(shape, jnp.float32)


def _pack_bf16_pairs(xs):
    xi = lax.bitcast_convert_type(xs, jnp.int32)
    r = lax.shift_right_logical(
        xi + 0x7FFF + lax.shift_right_logical(xi, 16) % 2, 16)
    return r[:, :128] | lax.shift_left(r[:, 128:], 16)


def _proj(x, wl, wr, c_chunks):
    n, k = x.shape
    d = wl.shape[1]

    def body(x_ref, wl_ref, wr_ref, xlc_ref, xr_ref, xlcb_ref, xrb_ref):
        xb = x_ref[...]
        xl = jnp.dot(xb, wl_ref[...], preferred_element_type=jnp.float32)
        xr = jnp.dot(xb, wr_ref[...], preferred_element_type=jnp.float32)
        xr_ref[...] = xr
        for c in range(c_chunks):
            xlc_ref[c] = xl[:, c * 128:(c + 1) * 128]
        for c2 in range(c_chunks // 2):
            blk = slice(c2 * 256, (c2 + 1) * 256)
            xlcb_ref[c2] = _pack_bf16_pairs(xl[:, blk])
            xrb_ref[:, c2 * 128:(c2 + 1) * 128] = _pack_bf16_pairs(xr[:, blk])

    return pl.pallas_call(
        body,
        grid=(n // BN,),
        in_specs=[
            pl.BlockSpec((BN, k), lambda i: (i, 0)),
            pl.BlockSpec((k, d), lambda i: (0, 0)),
            pl.BlockSpec((k, d), lambda i: (0, 0)),
        ],
        out_specs=[
            pl.BlockSpec((c_chunks, BN, 128), lambda i: (0, i, 0)),
            pl.BlockSpec((BN, d), lambda i: (i, 0)),
            pl.BlockSpec((c_chunks // 2, BN, 128), lambda i: (0, i, 0)),
            pl.BlockSpec((BN, d // 2), lambda i: (i, 0)),
        ],
        out_shape=[_f32(c_chunks, n, 128), _f32(n, d),
                   jax.ShapeDtypeStruct((c_chunks // 2, n, 128), jnp.int32),
                   jax.ShapeDtypeStruct((n, d // 2), jnp.int32)],
    )(x, wl, wr)


def _edge_transform(ea, we):
    e, de = ea.shape
    d = we.shape[1]

    def body(ea_ref, we_ref, ew_ref):
        ew = jnp.dot(ea_ref[...], we_ref[...],
                     preferred_element_type=jnp.float32)
        for c2 in range(d // 256):
            blk = slice(c2 * 256, (c2 + 1) * 256)
            ew_ref[:, c2 * 128:(c2 + 1) * 128] = _pack_bf16_pairs(ew[:, blk])

    return pl.pallas_call(
        body,
        grid=(e // BE_TC,),
        in_specs=[
            pl.BlockSpec((BE_TC, de), lambda i: (i, 0)),
            pl.BlockSpec((de, d), lambda i: (0, 0)),
        ],
        out_specs=pl.BlockSpec((BE_TC, d // 2), lambda i: (i, 0)),
        out_shape=jax.ShapeDtypeStruct((e, d // 2), jnp.int32),
    )(ea, we)


def _loop_attr(attr_p, deg_p, de):
    def body(a_ref, d_ref, o_ref):
        deg = jnp.sum(d_ref[...], axis=0)
        asum = jnp.sum(a_ref[...], axis=0)[:, :de]
        o_ref[...] = asum / jnp.maximum(deg, 1.0)[:, None]

    return pl.pallas_call(
        body,
        grid=(NP // BN,),
        in_specs=[
            pl.BlockSpec((2, BN, 128), lambda i: (0, i, 0)),
            pl.BlockSpec((NW, BN), lambda i: (0, i)),
        ],
        out_specs=pl.BlockSpec((BN, de), lambda i: (i, 0)),
        out_shape=_f32(NP, de),
    )(attr_p, deg_p)


def _p_self(xlc, xr, la, we, att2d, c_chunks):
    d = xr.shape[1]
    de = la.shape[1]

    def body(xlc_ref, xr_ref, la_ref, we_ref, att_ref, o_ref):
        lw = jnp.dot(la_ref[...], we_ref[...],
                     preferred_element_type=jnp.float32)
        acc = jnp.zeros((BN,), jnp.float32)
        for c in range(c_chunks):
            sl = slice(c * 128, (c + 1) * 128)
            v = xlc_ref[c] + xr_ref[:, sl] + lw[:, sl]
            v = jnp.maximum(v, 0.2 * v)
            acc = acc + jnp.sum(v * att_ref[0, sl][None, :], axis=1)
        o_ref[...] = jnp.exp(acc)[:, None]

    return pl.pallas_call(
        body,
        grid=(NP // BN,),
        in_specs=[
            pl.BlockSpec((c_chunks, BN, 128), lambda i: (0, i, 0)),
            pl.BlockSpec((BN, d), lambda i: (i, 0)),
            pl.BlockSpec((BN, de), lambda i: (i, 0)),
            pl.BlockSpec((de, d), lambda i: (0, 0)),
            pl.BlockSpec((1, d), lambda i: (0, 0)),
        ],
        out_specs=pl.BlockSpec((BN, 1), lambda i: (i, 0)),
        out_shape=_f32(NP, 1),
    )(xlc, xr, la, we, att2d)


def _combine(ou_list, xlc, s_p, p_self, b2d, c_chunks, relu):
    d = c_chunks * 128

    def body(*refs):
        ou_refs = refs[:c_chunks]
        xlc_ref, s_ref, ps_ref, b_ref, o_ref = refs[c_chunks:]
        den = jnp.sum(s_ref[...], axis=0)[:, None] + ps_ref[...] + 1e-16
        for c in range(c_chunks):
            num = ou_refs[c][0] + ou_refs[c][1] + ps_ref[...] * xlc_ref[c]
            val = num / den + b_ref[0, c * 128:(c + 1) * 128][None, :]
            if relu:
                val = jnp.maximum(val, 0.0)
            o_ref[:, c * 128:(c + 1) * 128] = val

    return pl.pallas_call(
        body,
        grid=(NP // BN,),
        in_specs=[pl.BlockSpec((2, BN, 128), lambda i: (0, i, 0))
                  for _ in range(c_chunks)] + [
            pl.BlockSpec((c_chunks, BN, 128), lambda i: (0, i, 0)),
            pl.BlockSpec((NW, BN), lambda i: (0, i)),
            pl.BlockSpec((BN, 1), lambda i: (i, 0)),
            pl.BlockSpec((1, d), lambda i: (0, 0)),
        ],
        out_specs=pl.BlockSpec((BN, d), lambda i: (i, 0)),
        out_shape=_f32(NP, d),
    )(*ou_list, xlc, s_p, p_self, b2d)


def _zero_1d(ref, n):
    def zb(i, _):
        ref[pl.ds(i * LANES, LANES)] = jnp.zeros((LANES,), jnp.float32)
        return 0
    lax.fori_loop(0, n // LANES, zb, 0)


def _edge_blocks(nblocks, w, body_fn):
    kmax = -(-nblocks // NW)

    def blk(kk, _):
        b = kk * NW + w

        @pl.when(b < nblocks)
        def _():
            body_fn(b * BE)
        return 0

    lax.fori_loop(0, kmax, blk, 0)


def _make_pass0(e, de):
    mesh = plsc.VectorSubcoreMesh(core_axis_name="c", subcore_axis_name="s", num_cores=NC, num_subcores=NS)
    rows_per_tile = NP // NS
    zrows = 32

    @functools.partial(
        pl.kernel,
        out_type=(_f32(2, NP, 128), _f32(NW, NP)),
        mesh=mesh,
        compiler_params=pltpu.CompilerParams(needs_layout_passes=False),
        scratch_types=(
            pltpu.VMEM((BE,), jnp.int32),
            pltpu.VMEM((BE, de), jnp.float32),
            pltpu.VMEM((BE, 128), jnp.float32),
            pltpu.VMEM((NP,), jnp.float32),
            pltpu.VMEM((zrows, 128), jnp.float32),
            pltpu.VMEM_SHARED((NP, 128), jnp.float32),
            pltpu.SemaphoreType.DMA,
        ),
    )
    def k(dst_h, ea_h, attr_h, deg_h, dstb, eab, rows, dega, ztile, acc, sem):
        c_ax = lax.axis_index("c")
        s_ax = lax.axis_index("s")
        w = s_ax * NC + c_ax

        _zero_1d(dega, NP)

        def zrow(i, _):
            for j in range(128 // LANES):
                ztile[i, pl.ds(j * LANES, LANES)] = jnp.zeros(
                    (LANES,), jnp.float32)
            return 0
        lax.fori_loop(0, zrows, zrow, 0)

        def zrow2(i, _):
            for j in range(128 // LANES):
                rows[i, pl.ds(j * LANES, LANES)] = jnp.zeros(
                    (LANES,), jnp.float32)
            return 0
        lax.fori_loop(0, BE, zrow2, 0)

        def zcp(q, _):
            pltpu.sync_copy(
                ztile,
                acc.at[pl.ds(s_ax * rows_per_tile + q * zrows, zrows)])
            return 0
        lax.fori_loop(0, rows_per_tile // zrows, zcp, 0)
        plsc.subcore_barrier()

        def do_block(base):
            pltpu.sync_copy(dst_h.at[pl.ds(base, BE)], dstb)
            pltpu.sync_copy(ea_h.at[pl.ds(base, BE)], eab)

            def crow(i, _):
                rows[i, pl.ds(0, LANES)] = eab[i, pl.ds(0, LANES)]
                return 0
            lax.fori_loop(0, BE, crow, 0)
            pltpu.sync_copy(rows, acc.at[dstb], add=True)
            ones = jnp.ones((LANES,), jnp.float32)
            for jj in range(BE // LANES):
                idx = dstb[pl.ds(jj * LANES, LANES)]
                plsc.addupdate_scatter(dega, [idx], ones)

        _edge_blocks(e // BE, w, do_block)

        pltpu.sync_copy(dega, deg_h.at[w])
        plsc.subcore_barrier()
        sl = pl.ds(s_ax * rows_per_tile, rows_per_tile)
        pltpu.sync_copy(acc.at[sl], attr_h.at[c_ax, sl])

    return k


def _make_pass1(e, d, c_chunks):
    mesh = plsc.VectorSubcoreMesh(core_axis_name="c", subcore_axis_name="s", num_cores=NC, num_subcores=NS)
    be = 64
    nb = e // be
    kmax = -(-nb // NW)
    c2n = c_chunks // 2

    buf_set = tuple(
        (pltpu.VMEM((be,), jnp.int32),
         pltpu.VMEM((be,), jnp.int32),
         pltpu.VMEM((be, d // 2), jnp.int32),
         pltpu.VMEM((be, d // 2), jnp.int32),
         tuple(pltpu.VMEM((be, 128), jnp.int32)
               for _ in range(c2n)),
         pltpu.SemaphoreType.DMA)
        for _ in range(2))

    @functools.partial(
        pl.kernel,
        out_type=(_f32(e), _f32(NW, NP)),
        mesh=mesh,
        compiler_params=pltpu.CompilerParams(needs_layout_passes=False),
        scratch_types=(
            buf_set,
            pltpu.VMEM((d,), jnp.float32),
            pltpu.VMEM((be,), jnp.float32),
            pltpu.VMEM((be,), jnp.float32),
            pltpu.VMEM((NP,), jnp.float32),
        ),
    )
    def k(src_h, dst_h, xr_h, ew_h, att_h, *rest):
        xl_hs = rest[:c2n]
        p_h, s_h = rest[c2n:c2n + 2]
        bufs, attv, lblock, pblock, sacc = rest[c2n + 2:]

        c_ax = lax.axis_index("c")
        s_ax = lax.axis_index("s")
        w = s_ax * NC + c_ax

        pltpu.sync_copy(att_h, attv)
        _zero_1d(sacc, NP)

        lane_iota = lax.iota(jnp.int32, LANES)

        def issue(t, b):
            srcb, dstb, ewb, xrb, xlbs, sem = bufs[t]
            base = b * be
            pltpu.sync_copy(src_h.at[pl.ds(base, be)], srcb)
            pltpu.sync_copy(dst_h.at[pl.ds(base, be)], dstb)
            pltpu.async_copy(ew_h.at[pl.ds(base, be)], ewb, sem)
            pltpu.async_copy(xr_h.at[dstb], xrb, sem)
            for c2 in range(c2n):
                pltpu.async_copy(xl_hs[c2].at[srcb], xlbs[c2], sem)

        def compute(t, b):
            srcb, dstb, ewb, xrb, xlbs, sem = bufs[t]
            base = b * be
            pltpu.make_async_copy(ew_h.at[pl.ds(0, be)], ewb, sem).wait()
            pltpu.make_async_copy(xr_h.at[dstb], xrb, sem).wait()
            for c2 in range(c2n):
                pltpu.make_async_copy(xl_hs[c2].at[srcb], xlbs[c2],
                                      sem).wait()

            himask = jnp.full((LANES,), -65536, jnp.int32)

            def lo16(x):
                return plsc.bitcast(lax.shift_left(x, 16), jnp.float32)

            def hi16(x):
                return plsc.bitcast(jnp.bitwise_and(x, himask), jnp.float32)

            def edge(i, lvec):
                acc = jnp.zeros((LANES,), jnp.float32)
                for c2 in range(c2n):
                    for j2 in range(128 // LANES):
                        col = c2 * 128 + j2 * LANES
                        f_lo = c2 * 256 + j2 * LANES
                        xl32 = xlbs[c2][i, pl.ds(j2 * LANES, LANES)]
                        xr32 = xrb[i, pl.ds(col, LANES)]
                        ew32 = ewb[i, pl.ds(col, LANES)]
                        va = lo16(xl32) + lo16(xr32) + lo16(ew32)
                        va = jnp.maximum(va, 0.2 * va)
                        acc = acc + va * attv[pl.ds(f_lo, LANES)]
                        vb = hi16(xl32) + hi16(xr32) + hi16(ew32)
                        vb = jnp.maximum(vb, 0.2 * vb)
                        acc = acc + vb * attv[pl.ds(f_lo + 128, LANES)]
                lsum = jnp.sum(acc)
                lvec = jnp.where(lane_iota == i % LANES, lsum, lvec)

                @pl.when(i % LANES == LANES - 1)
                def _():
                    lblock[pl.ds(i - (LANES - 1), LANES)] = lvec
                return lvec

            lax.fori_loop(0, be, edge, jnp.zeros((LANES,), jnp.float32))

            for jj in range(be // LANES):
                pv = jnp.exp(lblock[pl.ds(jj * LANES, LANES)])
                pblock[pl.ds(jj * LANES, LANES)] = pv
                idx = dstb[pl.ds(jj * LANES, LANES)]
                plsc.addupdate_scatter(sacc, [idx], pv)
            pltpu.sync_copy(pblock, p_h.at[pl.ds(base, be)])

        def blk_of(j):
            return j * NW + w

        @pl.when(blk_of(0) < nb)
        def _():
            issue(0, blk_of(0))

        def pair(kk, _):
            j0 = 2 * kk

            @pl.when(blk_of(j0 + 1) < nb)
            def _():
                issue(1, blk_of(j0 + 1))

            @pl.when(blk_of(j0) < nb)
            def _():
                compute(0, blk_of(j0))

            @pl.when(blk_of(j0 + 2) < nb)
            def _():
                issue(0, blk_of(j0 + 2))

            @pl.when(blk_of(j0 + 1) < nb)
            def _():
                compute(1, blk_of(j0 + 1))
            return 0

        lax.fori_loop(0, (kmax + 1) // 2, pair, 0)
        pltpu.sync_copy(sacc, s_h.at[w])

    return k


def _make_pass2(e, c_chunks):
    mesh = plsc.VectorSubcoreMesh(core_axis_name="c", subcore_axis_name="s", num_cores=NC, num_subcores=NS)
    rows_per_tile = NP // NS
    zrows = 32

    @functools.partial(
        pl.kernel,
        out_type=tuple(_f32(2, NP, 128) for _ in range(c_chunks)),
        mesh=mesh,
        compiler_params=pltpu.CompilerParams(needs_layout_passes=False),
        scratch_types=(
            tuple((pltpu.VMEM((BE,), jnp.int32),
                   pltpu.VMEM((BE,), jnp.int32),
                   pltpu.VMEM((BE,), jnp.float32),
                   pltpu.VMEM((BE, 128), jnp.float32),
                   pltpu.SemaphoreType.DMA)
                  for _ in range(2)),
            pltpu.VMEM((zrows, 128), jnp.float32),
            pltpu.VMEM_SHARED((NP, 128), jnp.float32),
        ),
    )
    def k(src_h, dst_h, p_h, *rest):
        xl_hs = rest[:c_chunks]
        ou_hs = rest[c_chunks:2 * c_chunks]
        bufs, ztile, acc = rest[2 * c_chunks:]

        c_ax = lax.axis_index("c")
        s_ax = lax.axis_index("s")
        w = s_ax * NC + c_ax

        def zrow(i, _):
            for j in range(128 // LANES):
                ztile[i, pl.ds(j * LANES, LANES)] = jnp.zeros(
                    (LANES,), jnp.float32)
            return 0
        lax.fori_loop(0, zrows, zrow, 0)

        nb = e // BE
        kmax = -(-nb // NW)
        lane_iota = lax.iota(jnp.int32, LANES)

        def blk_of(j):
            return j * NW + w

        for c in range(c_chunks):
            def zcp(q, _):
                pltpu.sync_copy(
                    ztile,
                    acc.at[pl.ds(s_ax * rows_per_tile + q * zrows, zrows)])
                return 0
            lax.fori_loop(0, rows_per_tile // zrows, zcp, 0)
            plsc.subcore_barrier()

            def issue(t, b):
                srcb, dstb, pb, rows, sem = bufs[t]
                base = b * BE
                pltpu.sync_copy(src_h.at[pl.ds(base, BE)], srcb)
                pltpu.sync_copy(dst_h.at[pl.ds(base, BE)], dstb)
                pltpu.sync_copy(p_h.at[pl.ds(base, BE)], pb)
                pltpu.async_copy(xl_hs[c].at[srcb], rows, sem)

            def proc(t, b):
                srcb, dstb, pb, rows, sem = bufs[t]
                pltpu.make_async_copy(xl_hs[c].at[srcb], rows, sem).wait()

                def edge(i, _):
                    pv = plsc.load_gather(pb, [lane_iota * 0 + i])
                    for j in range(128 // LANES):
                        sl = pl.ds(j * LANES, LANES)
                        rows[i, sl] = rows[i, sl] * pv
                    return 0

                lax.fori_loop(0, BE, edge, 0)
                pltpu.sync_copy(rows, acc.at[dstb], add=True)

            @pl.when(blk_of(0) < nb)
            def _():
                issue(0, blk_of(0))

            def pair(kk, _):
                j0 = 2 * kk

                @pl.when(blk_of(j0 + 1) < nb)
                def _():
                    issue(1, blk_of(j0 + 1))

                @pl.when(blk_of(j0) < nb)
                def _():
                    proc(0, blk_of(j0))

                @pl.when(blk_of(j0 + 2) < nb)
                def _():
                    issue(0, blk_of(j0 + 2))

                @pl.when(blk_of(j0 + 1) < nb)
                def _():
                    proc(1, blk_of(j0 + 1))
                return 0

            lax.fori_loop(0, (kmax + 1) // 2, pair, 0)
            plsc.subcore_barrier()
            sl = pl.ds(s_ax * rows_per_tile, rows_per_tile)
            pltpu.sync_copy(acc.at[sl], ou_hs[c].at[c_ax, sl])
            plsc.subcore_barrier()

    return k


def _gat_layer(xin, src, dst, ea, la, wl, wr, we, att, b, relu):
    e = src.shape[0]
    d = wl.shape[1]
    c_chunks = d // 128

    xlc, xr, xlcb, xri = _proj(xin, wl, wr, c_chunks)
    ew = _edge_transform(ea, we)
    att2d = att.reshape(1, d)
    ps = _p_self(xlc, xr, la, we, att2d, c_chunks)

    xl_list = [xlc[c] for c in range(c_chunks)]
    xlb_list = [xlcb[c2] for c2 in range(c_chunks // 2)]
    if _DEBUG_JNP_PASS1:
        xl_full = jnp.concatenate(xl_list, axis=1)
        v = xl_full[src] + xr[dst] + ew
        logit = jnp.sum(jnp.maximum(v, 0.2 * v) * att[None, :], axis=1)
        p = jnp.exp(logit)
        s_p = jnp.zeros((NW, NP), jnp.float32)
        s_p = s_p.at[0].set(jax.ops.segment_sum(p, dst, num_segments=NP))
    else:
        p, s_p = _make_pass1(e, d, c_chunks)(src, dst, xri, ew, att,
                                             *xlb_list)
    if _DEBUG_JNP_PASS2:
        ou_list = []
        for c in range(c_chunks):
            ou = jax.ops.segment_sum(p[:, None] * xl_list[c][src], dst,
                                     num_segments=NP)
            ou_list.append(jnp.stack([ou, jnp.zeros_like(ou)]))
    else:
        ou_list = _make_pass2(e, c_chunks)(src, dst, p, *xl_list)
    if not isinstance(ou_list, (list, tuple)):
        ou_list = [ou_list]
    return _combine(list(ou_list), xlc, s_p, ps, b.reshape(1, d),
                    c_chunks, relu)


def kernel(x, edge_index, edge_attr, Wl1, Wr1, We1, att1, b1,
           Wl2, Wr2, We2, att2, b2):
    n = x.shape[0]
    e = edge_index.shape[1]
    src = edge_index[0].astype(jnp.int32)
    dst = edge_index[1].astype(jnp.int32)
    ea = edge_attr.astype(jnp.float32)
    xp = jnp.pad(x.astype(jnp.float32), ((0, NP - n), (0, 0)))

    attr_p, deg_p = _make_pass0(e, ea.shape[1])(dst, ea)
    la = _loop_attr(attr_p, deg_p, ea.shape[1])

    h = _gat_layer(xp, src, dst, ea, la, Wl1, Wr1, We1, att1, b1, relu=True)
    out = _gat_layer(h, src, dst, ea, la, Wl2, Wr2, We2, att2, b2, relu=False)
    return out[:n]
